# baseline (speedup 1.0000x reference)
import functools

import jax
import jax.numpy as jnp
from jax import lax
from jax.experimental import pallas as pl
from jax.experimental.pallas import tpu as pltpu
from jax.experimental.pallas import tpu_sc as plsc

N_CLASSES = 1000000
EMBED_DIM = 32
BATCH = 16384

NC = 2
NS = 16
NW = NC * NS
B_PER_W = BATCH // NW
CHUNK = 128
N_CHUNKS = B_PER_W // CHUNK


def _gather_body(idx_hbm, table_hbm, out_hbm, idx_v, rows_v, sem):
    wid = lax.axis_index("s") * NC + lax.axis_index("c")
    base = wid * B_PER_W
    pltpu.sync_copy(idx_hbm.at[pl.ds(wid * N_CHUNKS, N_CHUNKS)], idx_v)
    copies = []
    for j in range(N_CHUNKS):
        copies.append(
            pltpu.async_copy(
                table_hbm.at[idx_v.at[j]],
                rows_v.at[pl.ds(j * CHUNK, CHUNK)],
                sem,
            )
        )
    for c in copies:
        c.wait()
    pltpu.sync_copy(rows_v, out_hbm.at[pl.ds(base, B_PER_W)])


@functools.partial(jax.jit, static_argnames=())
def kernel(batch, table):
    mesh = plsc.VectorSubcoreMesh(
        core_axis_name="c", subcore_axis_name="s",
        num_cores=NC, num_subcores=NS,
    )
    idx2 = batch.reshape(NW * N_CHUNKS, CHUNK)
    out = pl.kernel(
        _gather_body,
        out_type=jax.ShapeDtypeStruct((BATCH, EMBED_DIM), jnp.float32),
        mesh=mesh,
        scratch_types=[
            pltpu.VMEM((N_CHUNKS, CHUNK), jnp.int32),
            pltpu.VMEM((B_PER_W, EMBED_DIM), jnp.float32),
            pltpu.SemaphoreType.DMA,
        ],
        compiler_params=pltpu.CompilerParams(use_tc_tiling_on_sc=False),
    )(idx2, table)
    return out.reshape(BATCH, 1, EMBED_DIM)


# speedup vs baseline: 3.4496x; 3.4496x over previous
import functools

import jax
import jax.numpy as jnp
from jax import lax
from jax.experimental import pallas as pl
from jax.experimental.pallas import tpu as pltpu
from jax.experimental.pallas import tpu_sc as plsc

N_CLASSES = 1000000
EMBED_DIM = 32
BATCH = 16384

NC = 2
NS = 16
NW = NC * NS
B_PER_W = BATCH // NW
LANES = 128
RC = 16
ROUNDS = B_PER_W // RC


def _gather_body(idx_hbm, tableT_hbm, out_hbm, idx_v, tiles_v, out_stage,
                 sem):
    wid = lax.axis_index("s") * NC + lax.axis_index("c")
    base = wid * B_PER_W
    pltpu.sync_copy(idx_hbm.at[pl.ds(base, B_PER_W)], idx_v)

    lanes16 = lax.iota(jnp.int32, 16)

    def round_body(r, _):
        rv = idx_v[pl.ds(r * RC, RC)]
        blk = (rv >> 7) << 7
        copies = []
        for k in range(RC):
            start = pl.multiple_of(blk[k], LANES)
            copies.append(
                pltpu.async_copy(
                    tableT_hbm.at[:, pl.ds(start, LANES)],
                    tiles_v.at[k],
                    sem,
                )
            )
        for cp in copies:
            cp.wait()
        rlane = rv & 127
        for k in range(RC):
            lane_vec = jnp.full((16,), rlane[k], jnp.int32)
            lo = plsc.load_gather(
                tiles_v, [jnp.full((16,), k, jnp.int32), lanes16, lane_vec])
            hi = plsc.load_gather(
                tiles_v,
                [jnp.full((16,), k, jnp.int32), lanes16 + 16, lane_vec])
            out_stage[k, pl.ds(0, 16)] = lo
            out_stage[k, pl.ds(16, 16)] = hi
        row0 = pl.multiple_of(base + r * RC, RC)
        pltpu.sync_copy(out_stage, out_hbm.at[pl.ds(row0, RC)])
        return ()

    lax.fori_loop(0, ROUNDS, round_body, ())
    return


@jax.jit
def kernel(batch, table):
    mesh = plsc.VectorSubcoreMesh(
        core_axis_name="c", subcore_axis_name="s",
        num_cores=NC, num_subcores=NS,
    )
    out = pl.kernel(
        _gather_body,
        out_type=jax.ShapeDtypeStruct((BATCH, EMBED_DIM), jnp.float32),
        mesh=mesh,
        scratch_types=[
            pltpu.VMEM((B_PER_W,), jnp.int32),
            pltpu.VMEM((RC, EMBED_DIM, LANES), jnp.float32),
            pltpu.VMEM((RC, EMBED_DIM), jnp.float32),
            pltpu.SemaphoreType.DMA,
        ],
        compiler_params=pltpu.CompilerParams(needs_layout_passes=False),
    )(batch, table.T)
    return out.reshape(BATCH, 1, EMBED_DIM)


# speedup vs baseline: 3.4551x; 1.0016x over previous
import jax
import jax.numpy as jnp
from jax import lax
from jax.experimental import pallas as pl
from jax.experimental.pallas import tpu as pltpu
from jax.experimental.pallas import tpu_sc as plsc

N_CLASSES = 1000000
EMBED_DIM = 32
BATCH = 16384

NC = 2
NS = 16
NW = NC * NS
B_PER_W = BATCH // NW
LANES = 128
RC = 16
ROUNDS = B_PER_W // RC


def _gather_body(idx_hbm, tableT_hbm, out_hbm, idx_v, tiles_v, out_stage,
                 sem):
    wid = lax.axis_index("s") * NC + lax.axis_index("c")
    base = wid * B_PER_W
    pltpu.sync_copy(idx_hbm.at[pl.ds(base, B_PER_W)], idx_v)

    lanes16 = lax.iota(jnp.int32, 16)

    def round_body(r, _):
        rv = idx_v[pl.ds(r * RC, RC)]
        blk = (rv >> 7) << 7
        copies = []
        for k in range(RC):
            start = pl.multiple_of(blk[k], LANES)
            copies.append(
                pltpu.async_copy(
                    tableT_hbm.at[:, pl.ds(start, LANES)],
                    tiles_v.at[k],
                    sem,
                )
            )
        for cp in copies:
            cp.wait()
        rlane = rv & 127
        for k in range(RC):
            lane_vec = jnp.full((16,), rlane[k], jnp.int32)
            lo = plsc.load_gather(
                tiles_v, [jnp.full((16,), k, jnp.int32), lanes16, lane_vec])
            hi = plsc.load_gather(
                tiles_v,
                [jnp.full((16,), k, jnp.int32), lanes16 + 16, lane_vec])
            out_stage[k, pl.ds(0, 16)] = lo
            out_stage[k, pl.ds(16, 16)] = hi
        row0 = pl.multiple_of(base + r * RC, RC)
        pltpu.sync_copy(out_stage, out_hbm.at[pl.ds(row0, RC)])
        return ()

    lax.fori_loop(0, ROUNDS, round_body, ())
    return


@jax.jit
def kernel(batch, table):
    mesh = plsc.VectorSubcoreMesh(
        core_axis_name="c", subcore_axis_name="s",
        num_cores=NC, num_subcores=NS,
    )
    out = pl.kernel(
        _gather_body,
        out_type=jax.ShapeDtypeStruct((BATCH, EMBED_DIM), jnp.float32),
        mesh=mesh,
        scratch_types=[
            pltpu.VMEM((B_PER_W,), jnp.int32),
            pltpu.VMEM((RC, EMBED_DIM, LANES), jnp.float32),
            pltpu.VMEM((RC, EMBED_DIM), jnp.float32),
            pltpu.SemaphoreType.DMA,
        ],
        compiler_params=pltpu.CompilerParams(needs_layout_passes=False),
    )(batch, table.T)
    return out.reshape(BATCH, 1, EMBED_DIM)


# speedup vs baseline: 4.3588x; 1.2616x over previous
import jax
import jax.numpy as jnp
from jax import lax
from jax.experimental import pallas as pl
from jax.experimental.pallas import tpu as pltpu
from jax.experimental.pallas import tpu_sc as plsc

N_CLASSES = 1000000
EMBED_DIM = 32
BATCH = 16384

NC = 2
NS = 16
NW = NC * NS
B_PER_W = BATCH // NW
LANES = 128
RCP = 8
SUBS = 8
MACROS = B_PER_W // (RCP * SUBS)


def _gather_body(idx_hbm, tableT_hbm, out_hbm, idx_v, buf_a, buf_b,
                 out_stage, sem):
    wid = lax.axis_index("s") * NC + lax.axis_index("c")
    base = wid * B_PER_W
    pltpu.sync_copy(idx_hbm.at[pl.ds(base, B_PER_W)], idx_v)

    lanes16 = lax.iota(jnp.int32, 16)
    bufs = (buf_a, buf_b)

    def load_rv(m, local):
        return idx_v[pl.ds((m * (SUBS // 2) + local // 2) * 16, 16)]

    def fire(m, local):
        rv = load_rv(m, local)
        buf = bufs[local % 2]
        for k in range(RCP):
            lane = (local % 2) * RCP + k
            start = pl.multiple_of((rv[lane] >> 7) << 7, LANES)
            pltpu.async_copy(
                tableT_hbm.at[:, pl.ds(start, LANES)], buf.at[k], sem
            )

    def drain(local):
        buf = bufs[local % 2]
        for k in range(RCP):
            pltpu.make_async_copy(
                tableT_hbm.at[:, pl.ds(0, LANES)], buf.at[k], sem
            ).wait()

    def extract(m, local):
        rv = load_rv(m, local)
        buf = bufs[local % 2]
        for k in range(RCP):
            lane = (local % 2) * RCP + k
            lane_vec = jnp.full((16,), rv[lane] & 127, jnp.int32)
            kf = jnp.full((16,), k, jnp.int32)
            out_stage[k, pl.ds(0, 16)] = plsc.load_gather(
                buf, [kf, lanes16, lane_vec])
            out_stage[k, pl.ds(16, 16)] = plsc.load_gather(
                buf, [kf, lanes16 + 16, lane_vec])
        row0 = pl.multiple_of(base + (m * SUBS + local) * RCP, RCP)
        pltpu.sync_copy(out_stage, out_hbm.at[pl.ds(row0, RCP)])

    fire(0, 0)

    def macro_body(m, _):
        for local in range(SUBS):
            if local < SUBS - 1:
                fire(m, local + 1)
            else:
                @pl.when(m < MACROS - 1)
                def _():
                    fire(m + 1, 0)
            drain(local)
            extract(m, local)
        return ()

    lax.fori_loop(0, MACROS, macro_body, ())
    return


@jax.jit
def kernel(batch, table):
    mesh = plsc.VectorSubcoreMesh(
        core_axis_name="c", subcore_axis_name="s",
        num_cores=NC, num_subcores=NS,
    )
    out = pl.kernel(
        _gather_body,
        out_type=jax.ShapeDtypeStruct((BATCH, EMBED_DIM), jnp.float32),
        mesh=mesh,
        scratch_types=[
            pltpu.VMEM((B_PER_W,), jnp.int32),
            pltpu.VMEM((RCP, EMBED_DIM, LANES), jnp.float32),
            pltpu.VMEM((RCP, EMBED_DIM, LANES), jnp.float32),
            pltpu.VMEM((RCP, EMBED_DIM), jnp.float32),
            pltpu.SemaphoreType.DMA,
        ],
        compiler_params=pltpu.CompilerParams(needs_layout_passes=False),
    )(batch, table.T)
    return out.reshape(BATCH, 1, EMBED_DIM)


# speedup vs baseline: 4.3676x; 1.0020x over previous
import jax
import jax.numpy as jnp
from jax import lax
from jax.experimental import pallas as pl
from jax.experimental.pallas import tpu as pltpu
from jax.experimental.pallas import tpu_sc as plsc

N_CLASSES = 1000000
EMBED_DIM = 32
BATCH = 16384

NC = 2
NS = 16
NW = NC * NS
B_PER_W = BATCH // NW
LANES = 128
RCP = 4
SUBS = 16
NBUF = 4
MACROS = B_PER_W // (RCP * SUBS)


def _gather_body(idx_hbm, tableT_hbm, out_hbm, idx_v, buf_a, buf_b,
                 buf_c, buf_d, out_stage, sem):
    wid = lax.axis_index("s") * NC + lax.axis_index("c")
    base = wid * B_PER_W
    pltpu.sync_copy(idx_hbm.at[pl.ds(base, B_PER_W)], idx_v)

    lanes16 = lax.iota(jnp.int32, 16)
    bufs = (buf_a, buf_b, buf_c, buf_d)
    per_rv = 16 // RCP

    def load_rv(m, local):
        return idx_v[pl.ds((m * (SUBS // per_rv) + local // per_rv) * 16, 16)]

    def fire(m, local):
        rv = load_rv(m, local)
        buf = bufs[local % NBUF]
        for k in range(RCP):
            lane = (local % per_rv) * RCP + k
            start = pl.multiple_of((rv[lane] >> 7) << 7, LANES)
            pltpu.async_copy(
                tableT_hbm.at[:, pl.ds(start, LANES)], buf.at[k], sem
            )

    def drain(local):
        buf = bufs[local % NBUF]
        for k in range(RCP):
            pltpu.make_async_copy(
                tableT_hbm.at[:, pl.ds(0, LANES)], buf.at[k], sem
            ).wait()

    def extract(m, local):
        rv = load_rv(m, local)
        buf = bufs[local % NBUF]
        for k in range(RCP):
            lane = (local % per_rv) * RCP + k
            lane_vec = jnp.full((16,), rv[lane] & 127, jnp.int32)
            kf = jnp.full((16,), k, jnp.int32)
            out_stage[k, pl.ds(0, 16)] = plsc.load_gather(
                buf, [kf, lanes16, lane_vec])
            out_stage[k, pl.ds(16, 16)] = plsc.load_gather(
                buf, [kf, lanes16 + 16, lane_vec])
        row0 = pl.multiple_of(base + (m * SUBS + local) * RCP, RCP)
        pltpu.sync_copy(out_stage, out_hbm.at[pl.ds(row0, RCP)])

    for p in range(NBUF - 1):
        fire(0, p)

    def macro_body(m, _):
        for local in range(SUBS):
            nxt = local + NBUF - 1
            if nxt < SUBS:
                fire(m, nxt)
            else:
                @pl.when(m < MACROS - 1)
                def _():
                    fire(m + 1, nxt - SUBS)
            drain(local)
            extract(m, local)
        return ()

    lax.fori_loop(0, MACROS, macro_body, ())
    return


@jax.jit
def kernel(batch, table):
    mesh = plsc.VectorSubcoreMesh(
        core_axis_name="c", subcore_axis_name="s",
        num_cores=NC, num_subcores=NS,
    )
    out = pl.kernel(
        _gather_body,
        out_type=jax.ShapeDtypeStruct((BATCH, EMBED_DIM), jnp.float32),
        mesh=mesh,
        scratch_types=[
            pltpu.VMEM((B_PER_W,), jnp.int32),
            pltpu.VMEM((RCP, EMBED_DIM, LANES), jnp.float32),
            pltpu.VMEM((RCP, EMBED_DIM, LANES), jnp.float32),
            pltpu.VMEM((RCP, EMBED_DIM, LANES), jnp.float32),
            pltpu.VMEM((RCP, EMBED_DIM, LANES), jnp.float32),
            pltpu.VMEM((RCP, EMBED_DIM), jnp.float32),
            pltpu.SemaphoreType.DMA,
        ],
        compiler_params=pltpu.CompilerParams(needs_layout_passes=False),
    )(batch, table.T)
    return out.reshape(BATCH, 1, EMBED_DIM)
